# Initial kernel scaffold; baseline (speedup 1.0000x reference)
import jax
import jax.numpy as jnp
from jax.experimental import pallas as pl


def kernel(input_index, embeds):
    raise NotImplementedError("write your pallas kernel here")


# speedup vs baseline: 1.0233x; 1.0233x over previous
import functools

import jax
import jax.numpy as jnp
from jax import lax
from jax.experimental import pallas as pl
from jax.experimental.pallas import tpu as pltpu
from jax.experimental.pallas import tpu_sc as plsc

NUM_EMB = 1_000_000
DIMS = 32
B_TOTAL = 16384 * 50
NC, NS = 2, 16
NW = NC * NS
PER_W = B_TOTAL // NW
CW = 128
NCHUNK = PER_W // CW


def _make_gather():
    mesh = plsc.VectorSubcoreMesh(core_axis_name="c", subcore_axis_name="s")

    @functools.partial(
        pl.kernel,
        mesh=mesh,
        out_type=jax.ShapeDtypeStruct((B_TOTAL, DIMS), jnp.float32),
        scratch_types=[
            pltpu.VMEM((NCHUNK, CW), jnp.int32),
            pltpu.VMEM((CW, DIMS), jnp.float32),
            pltpu.SemaphoreType.DMA,
        ],
        compiler_params=pltpu.CompilerParams(use_tc_tiling_on_sc=False),
    )
    def gather_kernel(idx_hbm, table_hbm, out_hbm, idx_v, rows_v, sem):
        wid = lax.axis_index("s") * NC + lax.axis_index("c")
        base = wid * PER_W
        pltpu.sync_copy(idx_hbm.at[pl.ds(wid * NCHUNK, NCHUNK)], idx_v)

        def body(j, carry):
            pltpu.async_copy(table_hbm.at[idx_v.at[j]], rows_v, sem).wait()
            pltpu.sync_copy(rows_v, out_hbm.at[pl.ds(base + j * CW, CW)])
            return carry

        lax.fori_loop(0, NCHUNK, body, 0)

    return gather_kernel


_gather = _make_gather()


def kernel(input_index, embeds):
    idx = input_index.reshape(B_TOTAL // CW, CW).astype(jnp.int32)
    out = _gather(idx, embeds)
    return out.reshape(input_index.shape + (DIMS,))


# speedup vs baseline: 1.1115x; 1.0862x over previous
import functools

import jax
import jax.numpy as jnp
from jax import lax
from jax.experimental import pallas as pl
from jax.experimental.pallas import tpu as pltpu
from jax.experimental.pallas import tpu_sc as plsc

NUM_EMB = 1_000_000
DIMS = 32
B_TOTAL = 16384 * 50
NC, NS = 2, 16
NW = NC * NS
PER_W = B_TOTAL // NW
CW = 128
NCHUNK = PER_W // CW
NBUF = 8


def _make_gather():
    mesh = plsc.VectorSubcoreMesh(core_axis_name="c", subcore_axis_name="s")

    @functools.partial(
        pl.kernel,
        mesh=mesh,
        out_type=jax.ShapeDtypeStruct((B_TOTAL, DIMS), jnp.float32),
        scratch_types=[
            pltpu.VMEM((NCHUNK, CW), jnp.int32),
            pltpu.VMEM((NBUF, CW, DIMS), jnp.float32),
            pltpu.SemaphoreType.DMA((NBUF,)),
            pltpu.SemaphoreType.DMA((NBUF,)),
        ],
        compiler_params=pltpu.CompilerParams(use_tc_tiling_on_sc=False),
    )
    def gather_kernel(idx_hbm, table_hbm, out_hbm, idx_v, rows_v, gsem, ssem):
        wid = lax.axis_index("s") * NC + lax.axis_index("c")
        base = wid * PER_W
        pltpu.sync_copy(idx_hbm.at[pl.ds(wid * NCHUNK, NCHUNK)], idx_v)

        def burst(i, carry):
            g0 = i * NBUF
            gds = []
            for b in range(NBUF):
                @pl.when(i > 0)
                def _(b=b):
                    pltpu.make_async_copy(
                        rows_v.at[b], out_hbm.at[pl.ds(0, CW)], ssem.at[b]
                    ).wait()
                gds.append(
                    pltpu.async_copy(
                        table_hbm.at[idx_v.at[g0 + b]], rows_v.at[b], gsem.at[b]
                    )
                )
            for b in range(NBUF):
                gds[b].wait()
                pltpu.async_copy(
                    rows_v.at[b],
                    out_hbm.at[pl.ds(base + (g0 + b) * CW, CW)],
                    ssem.at[b],
                )
            return carry

        lax.fori_loop(0, NCHUNK // NBUF, burst, 0)
        for b in range(NBUF):
            pltpu.make_async_copy(
                rows_v.at[b], out_hbm.at[pl.ds(0, CW)], ssem.at[b]
            ).wait()

    return gather_kernel


_gather = _make_gather()


def kernel(input_index, embeds):
    idx = input_index.reshape(B_TOTAL // CW, CW).astype(jnp.int32)
    out = _gather(idx, embeds)
    return out.reshape(input_index.shape + (DIMS,))
